# baseline (speedup 1.0000x reference)
import functools

import numpy as np
import jax
import jax.numpy as jnp
from jax import lax
from jax.experimental import pallas as pl
from jax.experimental.pallas import tpu as pltpu
from jax.experimental.pallas import tpu_sc as plsc

N = 10000
NPAD = 10240
E = 320000
NG = 32
BN_EPS = 1e-5
INV_SQRT_BN = float(1.0 / np.sqrt(1.0 + BN_EPS))

NC = 2
NS = 16
CH = 125
ROWS_T = NPAD // NS

R = 2000
NB = N // R


def _sc_mesh():
    return plsc.VectorSubcoreMesh(core_axis_name="c", subcore_axis_name="s")


DEG_W = 16


def _deg_call(src3, zeros1, ones1):
    nch = E // (NC * NS) // CH

    @functools.partial(
        pl.kernel,
        mesh=_sc_mesh(),
        out_type=jax.ShapeDtypeStruct((NC, NPAD, DEG_W), jnp.float32),
        compiler_params=pltpu.CompilerParams(use_tc_tiling_on_sc=False),
        scratch_types=[
            pltpu.VMEM((nch, CH), jnp.int32),
            pltpu.VMEM((CH, DEG_W), jnp.float32),
            pltpu.VMEM_SHARED((NPAD, DEG_W), jnp.float32),
        ],
    )
    def deg_k(src_ref, z_ref, one_ref, out_ref, idx_v, ones_v, acc):
        c = lax.axis_index("c")
        s = lax.axis_index("s")
        w = c * NS + s
        pltpu.sync_copy(z_ref, acc.at[pl.ds(s * ROWS_T, ROWS_T)])
        pltpu.sync_copy(src_ref.at[w], idx_v)
        pltpu.sync_copy(one_ref, ones_v)
        plsc.subcore_barrier()

        def body(j, carry):
            pltpu.sync_copy(ones_v, acc.at[idx_v.at[j]], add=True)
            return carry

        lax.fori_loop(0, nch, body, 0)
        plsc.subcore_barrier()
        pltpu.sync_copy(acc.at[pl.ds(s * ROWS_T, ROWS_T)],
                        out_ref.at[c, pl.ds(s * ROWS_T, ROWS_T)])

    return deg_k(src3, zeros1, ones1)


def _spmm_call(tab, src3, dst3, zeros_d, d):
    nch = E // (NC * NS) // CH
    kb = 20
    nblk = nch // kb

    @functools.partial(
        pl.kernel,
        mesh=_sc_mesh(),
        out_type=jax.ShapeDtypeStruct((NC, NPAD, d), jnp.float32),
        compiler_params=pltpu.CompilerParams(use_tc_tiling_on_sc=False),
        scratch_types=[
            pltpu.VMEM((kb, CH), jnp.int32),
            pltpu.VMEM((kb, CH), jnp.int32),
            pltpu.VMEM((CH, d), jnp.float32),
            pltpu.VMEM((CH, d), jnp.float32),
            pltpu.VMEM_SHARED((NPAD, d), jnp.float32),
            pltpu.SemaphoreType.DMA,
            pltpu.SemaphoreType.DMA,
        ],
    )
    def spmm_k(tab_ref, src_ref, dst_ref, z_ref, out_ref,
               src_v, dst_v, rows_a, rows_b, acc, sem_a, sem_b):
        c = lax.axis_index("c")
        s = lax.axis_index("s")
        w = c * NS + s
        pltpu.sync_copy(z_ref, acc.at[pl.ds(s * ROWS_T, ROWS_T)])
        plsc.subcore_barrier()

        def blk_body(blk, carry):
            pltpu.sync_copy(src_ref.at[w, pl.ds(blk * kb, kb)], src_v)
            pltpu.sync_copy(dst_ref.at[w, pl.ds(blk * kb, kb)], dst_v)
            pltpu.async_copy(tab_ref.at[src_v.at[0]], rows_a, sem_a)

            def body(i, carry2):
                ja = 2 * i
                jb = ja + 1
                pltpu.async_copy(tab_ref.at[src_v.at[jb]], rows_b, sem_b)
                pltpu.make_async_copy(
                    tab_ref.at[src_v.at[ja]], rows_a, sem_a).wait()
                pltpu.sync_copy(rows_a, acc.at[dst_v.at[ja]], add=True)

                @pl.when(jb + 1 < kb)
                def _():
                    pltpu.async_copy(tab_ref.at[src_v.at[jb + 1]],
                                     rows_a, sem_a)

                pltpu.make_async_copy(
                    tab_ref.at[src_v.at[jb]], rows_b, sem_b).wait()
                pltpu.sync_copy(rows_b, acc.at[dst_v.at[jb]], add=True)
                return carry2

            lax.fori_loop(0, kb // 2, body, 0)
            return carry

        lax.fori_loop(0, nblk, blk_body, 0)
        plsc.subcore_barrier()
        pltpu.sync_copy(acc.at[pl.ds(s * ROWS_T, ROWS_T)],
                        out_ref.at[c, pl.ds(s * ROWS_T, ROWS_T)])

    return spmm_k(tab, src3, dst3, zeros_d)


def _spmm_spmem_call(tab2, src2, dst2, zeros_h, d):
    h = d // 2
    nch = E // NS // CH
    kb = 20
    nblk = nch // kb

    @functools.partial(
        pl.kernel,
        mesh=_sc_mesh(),
        out_type=jax.ShapeDtypeStruct((NC, NPAD, h), jnp.float32),
        compiler_params=pltpu.CompilerParams(use_tc_tiling_on_sc=False),
        scratch_types=[
            pltpu.VMEM((kb, CH), jnp.int32),
            pltpu.VMEM((kb, CH), jnp.int32),
            pltpu.VMEM((CH, h), jnp.float32),
            pltpu.VMEM((CH, h), jnp.float32),
            pltpu.VMEM((ROWS_T, h), jnp.float32),
            pltpu.VMEM_SHARED((NPAD, h), jnp.float32),
            pltpu.VMEM_SHARED((NPAD, h), jnp.float32),
            pltpu.SemaphoreType.DMA,
            pltpu.SemaphoreType.DMA,
        ],
    )
    def spmm_k(tab_ref, src_ref, dst_ref, z_ref, out_ref,
               src_v, dst_v, rows_a, rows_b, stage_v, tab_s, acc,
               sem_a, sem_b):
        c = lax.axis_index("c")
        s = lax.axis_index("s")
        rs = pl.ds(s * ROWS_T, ROWS_T)
        pltpu.sync_copy(z_ref, acc.at[rs])
        pltpu.sync_copy(tab_ref.at[c, rs], stage_v)
        pltpu.sync_copy(stage_v, tab_s.at[rs])
        plsc.subcore_barrier()

        def blk_body(blk, carry):
            pltpu.sync_copy(src_ref.at[s, pl.ds(blk * kb, kb)], src_v)
            pltpu.sync_copy(dst_ref.at[s, pl.ds(blk * kb, kb)], dst_v)
            pltpu.async_copy(tab_s.at[src_v.at[0]], rows_a, sem_a)

            def body(i, carry2):
                ja = 2 * i
                jb = ja + 1
                pltpu.async_copy(tab_s.at[src_v.at[jb]], rows_b, sem_b)
                pltpu.make_async_copy(
                    tab_s.at[src_v.at[ja]], rows_a, sem_a).wait()
                pltpu.sync_copy(rows_a, acc.at[dst_v.at[ja]], add=True)

                @pl.when(jb + 1 < kb)
                def _():
                    pltpu.async_copy(tab_s.at[src_v.at[jb + 1]],
                                     rows_a, sem_a)

                pltpu.make_async_copy(
                    tab_s.at[src_v.at[jb]], rows_b, sem_b).wait()
                pltpu.sync_copy(rows_b, acc.at[dst_v.at[jb]], add=True)
                return carry2

            lax.fori_loop(0, kb // 2, body, 0)
            return carry

        lax.fori_loop(0, nblk, blk_body, 0)
        plsc.subcore_barrier()
        pltpu.sync_copy(acc.at[rs], out_ref.at[c, rs])

    return spmm_k(tab2, src2, dst2, zeros_h)


def _row_spec(d, third=None):
    if third is None:
        return pl.BlockSpec((R, d), lambda i: (i, 0))
    return pl.BlockSpec((NC, R, d), lambda i: (0, i, 0))


def _full_spec(shape):
    nd = len(shape)
    return pl.BlockSpec(shape, lambda i, _n=nd: (0,) * _n)


def _prep_call(deg2, x):
    def body(deg_ref, x_ref, dinv_ref, xp_ref):
        dsum = deg_ref[0, :, 0:1] + deg_ref[1, :, 0:1]
        dinv = jnp.where(dsum > 0.0, lax.rsqrt(jnp.maximum(dsum, 1.0)), 0.0)
        dinv_ref[...] = dinv
        xp_ref[...] = x_ref[...] * dinv

    return pl.pallas_call(
        body,
        grid=(NB,),
        in_specs=[_row_spec(DEG_W, third=True), _row_spec(128)],
        out_specs=[_row_spec(1), _row_spec(128)],
        out_shape=[
            jax.ShapeDtypeStruct((N, 1), jnp.float32),
            jax.ShapeDtypeStruct((NPAD, 128), jnp.float32),
        ],
    )(deg2, x)


def _mid1_call(u1, dinv, x, w1):
    def body(u_ref, dinv_ref, x_ref, w1_ref, o1_ref, yp_ref):
        dinv = dinv_ref[...]
        t1 = (u_ref[0] + u_ref[1]) * (-dinv)
        o1_ref[...] = (
            jnp.dot(x_ref[...], w1_ref[0], preferred_element_type=jnp.float32)
            + jnp.dot(t1, w1_ref[1], preferred_element_type=jnp.float32))
        yp_ref[...] = t1 * dinv

    return pl.pallas_call(
        body,
        grid=(NB,),
        in_specs=[_row_spec(128, third=True), _row_spec(1), _row_spec(128),
                  _full_spec((3, 128, 128))],
        out_specs=[_row_spec(128), _row_spec(128)],
        out_shape=[
            jax.ShapeDtypeStruct((N, 128), jnp.float32),
            jax.ShapeDtypeStruct((NPAD, 128), jnp.float32),
        ],
    )(u1, dinv, x, w1)


def _mid2_call(u2, dinv, x, o1p, w1, b1, gam, bet, w2, b2):
    def body(u_ref, dinv_ref, x_ref, o1_ref, w1_ref, b1_ref, g_ref, be_ref,
             w2_ref, b2_ref, o2_ref, ab_ref):
        dinv = dinv_ref[...]
        t2 = (u_ref[0] + u_ref[1]) * (-2.0 * dinv) - x_ref[...]
        pre = (o1_ref[...]
               + jnp.dot(t2, w1_ref[2], preferred_element_type=jnp.float32)
               + b1_ref[...])
        h = jnp.maximum(pre, 0.0)
        h = h * (g_ref[...] * INV_SQRT_BN) + be_ref[...]
        a = jnp.dot(h, w2_ref[1], preferred_element_type=jnp.float32)
        b = jnp.dot(h, w2_ref[2], preferred_element_type=jnp.float32)
        o2_ref[...] = (
            jnp.dot(h, w2_ref[0], preferred_element_type=jnp.float32)
            - b + b2_ref[...])
        ab_ref[...] = jnp.concatenate([a, b], axis=1) * dinv

    return pl.pallas_call(
        body,
        grid=(NB,),
        in_specs=[_row_spec(128, third=True), _row_spec(1), _row_spec(128),
                  _row_spec(128), _full_spec((3, 128, 128)),
                  _full_spec((1, 128)), _full_spec((1, 128)),
                  _full_spec((1, 128)), _full_spec((3, 128, 64)),
                  _full_spec((1, 64))],
        out_specs=[_row_spec(64), _row_spec(128)],
        out_shape=[
            jax.ShapeDtypeStruct((N, 64), jnp.float32),
            jax.ShapeDtypeStruct((NPAD, 128), jnp.float32),
        ],
    )(u2, dinv, x, o1p, w1, b1, gam, bet, w2, b2)


def _mid3_call(uab, dinv, o2p):
    def body(u_ref, dinv_ref, o2_ref, o2b_ref, cp_ref):
        dinv = dinv_ref[...]
        usum = u_ref[0] + u_ref[1]
        o2b_ref[...] = o2_ref[...] - dinv * usum[:, :64]
        cpf = -(dinv * dinv) * usum[:, 64:]
        cp_ref[...] = jnp.stack([cpf[:, :32], cpf[:, 32:]])

    return pl.pallas_call(
        body,
        grid=(NB,),
        in_specs=[_row_spec(128, third=True), _row_spec(1), _row_spec(64)],
        out_specs=[_row_spec(64), _row_spec(32, third=True)],
        out_shape=[
            jax.ShapeDtypeStruct((N, 64), jnp.float32),
            jax.ShapeDtypeStruct((NC, NPAD, 32), jnp.float32),
        ],
    )(uab, dinv, o2p)


def _final_call(uc, dinv, o2b, batch2, wlin, blin):
    def body(uc_ref, dinv_ref, o2b_ref, b_ref, wl_ref, bl_ref, out_ref,
             s_ref, c_ref, m_ref):
        i = pl.program_id(0)

        @pl.when(i == 0)
        def _():
            s_ref[...] = jnp.zeros_like(s_ref)
            c_ref[...] = jnp.zeros_like(c_ref)
            m_ref[...] = jnp.full_like(m_ref, -1e30)

        dinv = dinv_ref[...]
        usum_c = jnp.concatenate([uc_ref[0], uc_ref[1]], axis=1)
        llb = usum_c * (-2.0 * dinv)
        h2 = jnp.maximum(o2b_ref[...] + llb, 0.0)
        b = b_ref[...]
        gids = lax.broadcasted_iota(jnp.int32, (1, NG), 1)
        onehot = (b == gids).astype(jnp.float32)
        s_ref[...] += lax.dot_general(
            onehot, h2, (((0,), (0,)), ((), ())),
            preferred_element_type=jnp.float32)
        ones = jnp.ones((R, 1), jnp.float32)
        c_ref[...] += lax.dot_general(
            onehot, ones, (((0,), (0,)), ((), ())),
            preferred_element_type=jnp.float32)
        lo = b[0, 0]
        hi = b[R - 1, 0]

        def gmax(g, carry):
            vals = jnp.where(b == g, h2, -1e30)
            mg = jnp.max(vals, axis=0, keepdims=True)
            m_ref[pl.ds(g, 1), :] = jnp.maximum(m_ref[pl.ds(g, 1), :], mg)
            return carry

        lax.fori_loop(lo, hi + 1, gmax, 0)

        @pl.when(i == NB - 1)
        def _():
            s = s_ref[...]
            cnt = c_ref[...]
            mean = s / jnp.maximum(cnt, 1.0)
            mx = jnp.where(m_ref[...] > -1e29, m_ref[...], 0.0)
            pooled = jnp.concatenate([s, mean, mx], axis=1)
            out_ref[...] = (
                jnp.dot(pooled, wl_ref[...],
                        preferred_element_type=jnp.float32) + bl_ref[...])

    return pl.pallas_call(
        body,
        grid=(NB,),
        in_specs=[_row_spec(32, third=True), _row_spec(1), _row_spec(64),
                  _row_spec(1), _full_spec((192, 16)), _full_spec((1, 16))],
        out_specs=pl.BlockSpec((NG, 16), lambda i: (0, 0)),
        out_shape=jax.ShapeDtypeStruct((NG, 16), jnp.float32),
        scratch_shapes=[
            pltpu.VMEM((NG, 64), jnp.float32),
            pltpu.VMEM((NG, 1), jnp.float32),
            pltpu.VMEM((NG, 64), jnp.float32),
        ],
    )(uc, dinv, o2b, batch2, wlin, blin)


def kernel(x, edge_index, batch, W1, b1, bn_gamma, bn_beta, W2, b2, Wlin, blin):
    src = edge_index[0]
    dst = edge_index[1]
    src_d = src.reshape(NC * NS, E // (NC * NS) // CH, CH)
    dst_d = dst.reshape(NC * NS, E // (NC * NS) // CH, CH)

    zeros1 = jnp.zeros((ROWS_T, DEG_W), jnp.float32)
    ones1 = jnp.ones((CH, DEG_W), jnp.float32)
    zeros128 = jnp.zeros((ROWS_T, 128), jnp.float32)
    zeros32 = jnp.zeros((ROWS_T, 32), jnp.float32)
    src_s = src.reshape(NS, E // NS // CH, CH)
    dst_s = dst.reshape(NS, E // NS // CH, CH)

    deg2 = _deg_call(src_d, zeros1, ones1)
    dinv, xp = _prep_call(deg2, x)
    u1 = _spmm_call(xp, src_d, dst_d, zeros128, 128)
    o1p, yp = _mid1_call(u1, dinv, x, W1)
    u2 = _spmm_call(yp, src_d, dst_d, zeros128, 128)

    b1r = b1.reshape(1, 128)
    gam = bn_gamma.reshape(1, 128)
    bet = bn_beta.reshape(1, 128)
    b2r = b2.reshape(1, 64)
    o2p, ab = _mid2_call(u2, dinv, x, o1p, W1, b1r, gam, bet, W2, b2r)

    uab = _spmm_call(ab, src_d, dst_d, zeros128, 128)
    o2b, cp = _mid3_call(uab, dinv, o2p)
    uc = _spmm_spmem_call(cp, src_s, dst_s, zeros32, 64)

    batch2 = batch.reshape(N, 1)
    blr = blin.reshape(1, 16)
    return _final_call(uc, dinv, o2b, batch2, Wlin, blr)


# speedup vs baseline: 1.0292x; 1.0292x over previous
import functools

import numpy as np
import jax
import jax.numpy as jnp
from jax import lax
from jax.experimental import pallas as pl
from jax.experimental.pallas import tpu as pltpu
from jax.experimental.pallas import tpu_sc as plsc

N = 10000
NPAD = 10240
E = 320000
NG = 32
BN_EPS = 1e-5
INV_SQRT_BN = float(1.0 / np.sqrt(1.0 + BN_EPS))

NC = 2
NS = 16
CH = 125
ROWS_T = NPAD // NS

R = 2000
NB = N // R


def _sc_mesh():
    return plsc.VectorSubcoreMesh(core_axis_name="c", subcore_axis_name="s")


DEG_W = 16


def _deg_call(src3, zeros1, ones1):
    nch = E // (NC * NS) // CH

    @functools.partial(
        pl.kernel,
        mesh=_sc_mesh(),
        out_type=jax.ShapeDtypeStruct((NC, NPAD, DEG_W), jnp.float32),
        compiler_params=pltpu.CompilerParams(use_tc_tiling_on_sc=False),
        scratch_types=[
            pltpu.VMEM((nch, CH), jnp.int32),
            pltpu.VMEM((CH, DEG_W), jnp.float32),
            pltpu.VMEM_SHARED((NPAD, DEG_W), jnp.float32),
        ],
    )
    def deg_k(src_ref, z_ref, one_ref, out_ref, idx_v, ones_v, acc):
        c = lax.axis_index("c")
        s = lax.axis_index("s")
        w = c * NS + s
        pltpu.sync_copy(z_ref, acc.at[pl.ds(s * ROWS_T, ROWS_T)])
        pltpu.sync_copy(src_ref.at[w], idx_v)
        pltpu.sync_copy(one_ref, ones_v)
        plsc.subcore_barrier()

        def body(j, carry):
            pltpu.sync_copy(ones_v, acc.at[idx_v.at[j]], add=True)
            return carry

        lax.fori_loop(0, nch, body, 0)
        plsc.subcore_barrier()
        pltpu.sync_copy(acc.at[pl.ds(s * ROWS_T, ROWS_T)],
                        out_ref.at[c, pl.ds(s * ROWS_T, ROWS_T)])

    return deg_k(src3, zeros1, ones1)


def _spmm_call(tab, src3, dst3, zeros_d, d):
    nch = E // (NC * NS) // CH
    kb = 20
    nblk = nch // kb

    @functools.partial(
        pl.kernel,
        mesh=_sc_mesh(),
        out_type=jax.ShapeDtypeStruct((NC, NPAD, d), jnp.float32),
        compiler_params=pltpu.CompilerParams(use_tc_tiling_on_sc=False),
        scratch_types=[
            pltpu.VMEM((kb, CH), jnp.int32),
            pltpu.VMEM((kb, CH), jnp.int32),
            pltpu.VMEM((CH, d), jnp.float32),
            pltpu.VMEM((CH, d), jnp.float32),
            pltpu.VMEM_SHARED((NPAD, d), jnp.float32),
            pltpu.SemaphoreType.DMA,
            pltpu.SemaphoreType.DMA,
        ],
    )
    def spmm_k(tab_ref, src_ref, dst_ref, z_ref, out_ref,
               src_v, dst_v, rows_a, rows_b, acc, sem_a, sem_b):
        c = lax.axis_index("c")
        s = lax.axis_index("s")
        w = c * NS + s
        pltpu.sync_copy(z_ref, acc.at[pl.ds(s * ROWS_T, ROWS_T)])
        plsc.subcore_barrier()

        def blk_body(blk, carry):
            pltpu.sync_copy(src_ref.at[w, pl.ds(blk * kb, kb)], src_v)
            pltpu.sync_copy(dst_ref.at[w, pl.ds(blk * kb, kb)], dst_v)
            pltpu.async_copy(tab_ref.at[src_v.at[0]], rows_a, sem_a)

            def body(i, carry2):
                ja = 2 * i
                jb = ja + 1
                pltpu.async_copy(tab_ref.at[src_v.at[jb]], rows_b, sem_b)
                pltpu.make_async_copy(
                    tab_ref.at[src_v.at[ja]], rows_a, sem_a).wait()
                pltpu.sync_copy(rows_a, acc.at[dst_v.at[ja]], add=True)

                @pl.when(jb + 1 < kb)
                def _():
                    pltpu.async_copy(tab_ref.at[src_v.at[jb + 1]],
                                     rows_a, sem_a)

                pltpu.make_async_copy(
                    tab_ref.at[src_v.at[jb]], rows_b, sem_b).wait()
                pltpu.sync_copy(rows_b, acc.at[dst_v.at[jb]], add=True)
                return carry2

            lax.fori_loop(0, kb // 2, body, 0)
            return carry

        lax.fori_loop(0, nblk, blk_body, 0)
        plsc.subcore_barrier()
        pltpu.sync_copy(acc.at[pl.ds(s * ROWS_T, ROWS_T)],
                        out_ref.at[c, pl.ds(s * ROWS_T, ROWS_T)])

    return spmm_k(tab, src3, dst3, zeros_d)


def _row_spec(d, third=None):
    if third is None:
        return pl.BlockSpec((R, d), lambda i: (i, 0))
    return pl.BlockSpec((NC, R, d), lambda i: (0, i, 0))


def _full_spec(shape):
    nd = len(shape)
    return pl.BlockSpec(shape, lambda i, _n=nd: (0,) * _n)


def _prep_call(deg2, x):
    def body(deg_ref, x_ref, dinv_ref, xp_ref):
        dsum = deg_ref[0, :, 0:1] + deg_ref[1, :, 0:1]
        dinv = jnp.where(dsum > 0.0, lax.rsqrt(jnp.maximum(dsum, 1.0)), 0.0)
        dinv_ref[...] = dinv
        xp_ref[...] = x_ref[...] * dinv

    return pl.pallas_call(
        body,
        grid=(NB,),
        in_specs=[_row_spec(DEG_W, third=True), _row_spec(128)],
        out_specs=[_row_spec(1), _row_spec(128)],
        out_shape=[
            jax.ShapeDtypeStruct((N, 1), jnp.float32),
            jax.ShapeDtypeStruct((NPAD, 128), jnp.float32),
        ],
    )(deg2, x)


def _mid1_call(u1, dinv, x, w1):
    def body(u_ref, dinv_ref, x_ref, w1_ref, o1_ref, yp_ref):
        dinv = dinv_ref[...]
        t1 = (u_ref[0] + u_ref[1]) * (-dinv)
        o1_ref[...] = (
            jnp.dot(x_ref[...], w1_ref[0], preferred_element_type=jnp.float32)
            + jnp.dot(t1, w1_ref[1], preferred_element_type=jnp.float32))
        yp_ref[...] = t1 * dinv

    return pl.pallas_call(
        body,
        grid=(NB,),
        in_specs=[_row_spec(128, third=True), _row_spec(1), _row_spec(128),
                  _full_spec((3, 128, 128))],
        out_specs=[_row_spec(128), _row_spec(128)],
        out_shape=[
            jax.ShapeDtypeStruct((N, 128), jnp.float32),
            jax.ShapeDtypeStruct((NPAD, 128), jnp.float32),
        ],
    )(u1, dinv, x, w1)


def _mid2_call(u2, dinv, x, o1p, w1, b1, gam, bet, w2, b2):
    def body(u_ref, dinv_ref, x_ref, o1_ref, w1_ref, b1_ref, g_ref, be_ref,
             w2_ref, b2_ref, o2_ref, ab_ref):
        dinv = dinv_ref[...]
        t2 = (u_ref[0] + u_ref[1]) * (-2.0 * dinv) - x_ref[...]
        pre = (o1_ref[...]
               + jnp.dot(t2, w1_ref[2], preferred_element_type=jnp.float32)
               + b1_ref[...])
        h = jnp.maximum(pre, 0.0)
        h = h * (g_ref[...] * INV_SQRT_BN) + be_ref[...]
        a = jnp.dot(h, w2_ref[1], preferred_element_type=jnp.float32)
        b = jnp.dot(h, w2_ref[2], preferred_element_type=jnp.float32)
        o2_ref[...] = (
            jnp.dot(h, w2_ref[0], preferred_element_type=jnp.float32)
            - b + b2_ref[...])
        ab_ref[...] = jnp.concatenate([a, b], axis=1) * dinv

    return pl.pallas_call(
        body,
        grid=(NB,),
        in_specs=[_row_spec(128, third=True), _row_spec(1), _row_spec(128),
                  _row_spec(128), _full_spec((3, 128, 128)),
                  _full_spec((1, 128)), _full_spec((1, 128)),
                  _full_spec((1, 128)), _full_spec((3, 128, 64)),
                  _full_spec((1, 64))],
        out_specs=[_row_spec(64), _row_spec(128)],
        out_shape=[
            jax.ShapeDtypeStruct((N, 64), jnp.float32),
            jax.ShapeDtypeStruct((NPAD, 128), jnp.float32),
        ],
    )(u2, dinv, x, o1p, w1, b1, gam, bet, w2, b2)


def _mid3_call(uab, dinv, o2p):
    def body(u_ref, dinv_ref, o2_ref, o2b_ref, cp_ref):
        dinv = dinv_ref[...]
        usum = u_ref[0] + u_ref[1]
        o2b_ref[...] = o2_ref[...] - dinv * usum[:, :64]
        cp_ref[...] = -(dinv * dinv) * usum[:, 64:]

    return pl.pallas_call(
        body,
        grid=(NB,),
        in_specs=[_row_spec(128, third=True), _row_spec(1), _row_spec(64)],
        out_specs=[_row_spec(64), _row_spec(64)],
        out_shape=[
            jax.ShapeDtypeStruct((N, 64), jnp.float32),
            jax.ShapeDtypeStruct((NPAD, 64), jnp.float32),
        ],
    )(uab, dinv, o2p)


def _final_call(uc, dinv, o2b, batch2, wlin, blin):
    def body(uc_ref, dinv_ref, o2b_ref, b_ref, wl_ref, bl_ref, out_ref,
             s_ref, c_ref, m_ref):
        i = pl.program_id(0)

        @pl.when(i == 0)
        def _():
            s_ref[...] = jnp.zeros_like(s_ref)
            c_ref[...] = jnp.zeros_like(c_ref)
            m_ref[...] = jnp.full_like(m_ref, -1e30)

        dinv = dinv_ref[...]
        llb = (uc_ref[0] + uc_ref[1]) * (-2.0 * dinv)
        h2 = jnp.maximum(o2b_ref[...] + llb, 0.0)
        b = b_ref[...]
        gids = lax.broadcasted_iota(jnp.int32, (1, NG), 1)
        onehot = (b == gids).astype(jnp.float32)
        s_ref[...] += lax.dot_general(
            onehot, h2, (((0,), (0,)), ((), ())),
            preferred_element_type=jnp.float32)
        ones = jnp.ones((R, 1), jnp.float32)
        c_ref[...] += lax.dot_general(
            onehot, ones, (((0,), (0,)), ((), ())),
            preferred_element_type=jnp.float32)
        lo = b[0, 0]
        hi = b[R - 1, 0]

        def gmax(g, carry):
            vals = jnp.where(b == g, h2, -1e30)
            mg = jnp.max(vals, axis=0, keepdims=True)
            m_ref[pl.ds(g, 1), :] = jnp.maximum(m_ref[pl.ds(g, 1), :], mg)
            return carry

        lax.fori_loop(lo, hi + 1, gmax, 0)

        @pl.when(i == NB - 1)
        def _():
            s = s_ref[...]
            cnt = c_ref[...]
            mean = s / jnp.maximum(cnt, 1.0)
            mx = jnp.where(m_ref[...] > -1e29, m_ref[...], 0.0)
            pooled = jnp.concatenate([s, mean, mx], axis=1)
            out_ref[...] = (
                jnp.dot(pooled, wl_ref[...],
                        preferred_element_type=jnp.float32) + bl_ref[...])

    return pl.pallas_call(
        body,
        grid=(NB,),
        in_specs=[_row_spec(64, third=True), _row_spec(1), _row_spec(64),
                  _row_spec(1), _full_spec((192, 16)), _full_spec((1, 16))],
        out_specs=pl.BlockSpec((NG, 16), lambda i: (0, 0)),
        out_shape=jax.ShapeDtypeStruct((NG, 16), jnp.float32),
        scratch_shapes=[
            pltpu.VMEM((NG, 64), jnp.float32),
            pltpu.VMEM((NG, 1), jnp.float32),
            pltpu.VMEM((NG, 64), jnp.float32),
        ],
    )(uc, dinv, o2b, batch2, wlin, blin)


def kernel(x, edge_index, batch, W1, b1, bn_gamma, bn_beta, W2, b2, Wlin, blin):
    src = edge_index[0]
    dst = edge_index[1]
    src_d = src.reshape(NC * NS, E // (NC * NS) // CH, CH)
    dst_d = dst.reshape(NC * NS, E // (NC * NS) // CH, CH)

    zeros1 = jnp.zeros((ROWS_T, DEG_W), jnp.float32)
    ones1 = jnp.ones((CH, DEG_W), jnp.float32)
    zeros128 = jnp.zeros((ROWS_T, 128), jnp.float32)
    zeros64 = jnp.zeros((ROWS_T, 64), jnp.float32)

    deg2 = _deg_call(src_d, zeros1, ones1)
    dinv, xp = _prep_call(deg2, x)
    u1 = _spmm_call(xp, src_d, dst_d, zeros128, 128)
    o1p, yp = _mid1_call(u1, dinv, x, W1)
    u2 = _spmm_call(yp, src_d, dst_d, zeros128, 128)

    b1r = b1.reshape(1, 128)
    gam = bn_gamma.reshape(1, 128)
    bet = bn_beta.reshape(1, 128)
    b2r = b2.reshape(1, 64)
    o2p, ab = _mid2_call(u2, dinv, x, o1p, W1, b1r, gam, bet, W2, b2r)

    uab = _spmm_call(ab, src_d, dst_d, zeros128, 128)
    o2b, cp = _mid3_call(uab, dinv, o2p)
    uc = _spmm_call(cp, src_d, dst_d, zeros64, 64)

    batch2 = batch.reshape(N, 1)
    blr = blin.reshape(1, 16)
    return _final_call(uc, dinv, o2b, batch2, Wlin, blr)
